# baseline (speedup 1.0000x reference)
import functools

import jax
import jax.numpy as jnp
from jax import lax
from jax.experimental import pallas as pl
from jax.experimental.pallas import tpu as pltpu
from jax.experimental.pallas import tpu_sc as plsc

_VOCAB = 1000
_MAX_LEN = 20
_HIDDEN = 32
_BATCH = 1024
_IN_F = _MAX_LEN * _HIDDEN
_OUT_F = _MAX_LEN * _VOCAB

_NC, _NS = 2, 16
_NW = _NC * _NS
_B_TOTAL = _BATCH * _MAX_LEN
_B_PER_W = _B_TOTAL // _NW


_PAD_W = 128


def _sc_gather(table_pad, idx_flat):
    mesh = plsc.VectorSubcoreMesh(core_axis_name="c", subcore_axis_name="s")

    @functools.partial(
        pl.kernel,
        mesh=mesh,
        out_type=jax.ShapeDtypeStruct((_B_TOTAL, _PAD_W), jnp.float32),
        scratch_types=[
            pltpu.VMEM((_B_PER_W,), jnp.int32),
            pltpu.VMEM((_B_PER_W, _PAD_W), jnp.float32),
            pltpu.SemaphoreType.DMA,
        ],
    )
    def k(table_hbm, idx_hbm, out_hbm, idx_v, rows_v, sem):
        wid = lax.axis_index("s") * _NC + lax.axis_index("c")
        base = wid * _B_PER_W
        pltpu.sync_copy(idx_hbm.at[pl.ds(base, _B_PER_W)], idx_v)
        pltpu.async_copy(table_hbm.at[idx_v], rows_v, sem).wait()
        pltpu.sync_copy(rows_v, out_hbm.at[pl.ds(base, _B_PER_W)])

    return k(table_pad, idx_flat)


def _mm_body(rows_ref, w_ref, b_ref, out_hbm, flat_ref, obuf_ref, sem):
    t = pl.program_id(0)

    @pl.when(t == 0)
    def _():
        for k in range(_MAX_LEN):
            flat_ref[:, k * _HIDDEN:(k + 1) * _HIDDEN] = (
                rows_ref[:, k, :_HIDDEN].astype(jnp.bfloat16))

    w = w_ref[0].astype(jnp.bfloat16)
    acc = lax.dot_general(
        flat_ref[...], w,
        (((1,), (1,)), ((), ())),
        preferred_element_type=jnp.float32,
    )

    slot = lax.rem(t, 2)

    @pl.when(t >= 2)
    def _():
        pltpu.make_async_copy(
            obuf_ref.at[slot], out_hbm.at[:, t - 2, :], sem.at[slot]).wait()

    obuf_ref[slot] = acc + b_ref[t, 0, :][None, :]
    pltpu.make_async_copy(
        obuf_ref.at[slot], out_hbm.at[:, t, :], sem.at[slot]).start()

    @pl.when(t == _MAX_LEN - 1)
    def _():
        pltpu.make_async_copy(
            obuf_ref.at[1 - slot], out_hbm.at[:, t - 1, :],
            sem.at[1 - slot]).wait()
        pltpu.make_async_copy(
            obuf_ref.at[slot], out_hbm.at[:, t, :], sem.at[slot]).wait()


def _projection(rows3, W3, b3):
    return pl.pallas_call(
        _mm_body,
        grid=(_MAX_LEN,),
        in_specs=[
            pl.BlockSpec((_BATCH, _MAX_LEN, _PAD_W), lambda j: (0, 0, 0)),
            pl.BlockSpec((1, _VOCAB, _IN_F), lambda j: (j, 0, 0)),
            pl.BlockSpec((_MAX_LEN, 1, _VOCAB), lambda j: (0, 0, 0)),
        ],
        out_specs=pl.BlockSpec(memory_space=pl.ANY),
        out_shape=jax.ShapeDtypeStruct((_BATCH, _MAX_LEN, _VOCAB), jnp.float32),
        scratch_shapes=[
            pltpu.VMEM((_BATCH, _IN_F), jnp.bfloat16),
            pltpu.VMEM((2, _BATCH, _VOCAB), jnp.float32),
            pltpu.SemaphoreType.DMA((2,)),
        ],
        compiler_params=pltpu.CompilerParams(
            dimension_semantics=("arbitrary",),
        ),
    )(rows3, W3, b3)


def kernel(x, embed_table, W, b):
    idx = x.reshape(-1).astype(jnp.int32)
    table_pad = jnp.pad(embed_table, ((0, 0), (0, _PAD_W - _HIDDEN)))
    rows = _sc_gather(table_pad, idx)
    rows3 = rows.reshape(_BATCH, _MAX_LEN, _PAD_W)
    W3 = W.reshape(_MAX_LEN, _VOCAB, _IN_F)
    b3 = b.reshape(_MAX_LEN, 1, _VOCAB)
    return _projection(rows3, W3, b3)


# speedup vs baseline: 1.1382x; 1.1382x over previous
import functools

import jax
import jax.numpy as jnp
from jax import lax
from jax.experimental import pallas as pl
from jax.experimental.pallas import tpu as pltpu
from jax.experimental.pallas import tpu_sc as plsc

_VOCAB = 1000
_MAX_LEN = 20
_HIDDEN = 32
_BATCH = 1024
_IN_F = _MAX_LEN * _HIDDEN
_OUT_F = _MAX_LEN * _VOCAB

_NC, _NS = 2, 16
_NW = _NC * _NS
_B_TOTAL = _BATCH * _MAX_LEN
_B_PER_W = _B_TOTAL // _NW


_PAD_W = 128


_B_PER_SUBCORE = _BATCH // _NW
_HALF = 16


def _sc_gather(table_pad, idx_flat):
    mesh = plsc.VectorSubcoreMesh(core_axis_name="c", subcore_axis_name="s")

    @functools.partial(
        pl.kernel,
        mesh=mesh,
        out_type=jax.ShapeDtypeStruct((_BATCH, _IN_F), jnp.float32),
        scratch_types=[
            pltpu.VMEM((_B_PER_W,), jnp.int32),
            pltpu.VMEM((_B_PER_W, _PAD_W), jnp.float32),
            pltpu.VMEM((_B_PER_SUBCORE, _IN_F), jnp.float32),
            pltpu.SemaphoreType.DMA,
        ],
    )
    def k(table_hbm, idx_hbm, out_hbm, idx_v, rows_v, comp_v, sem):
        wid = lax.axis_index("s") * _NC + lax.axis_index("c")
        base = wid * _B_PER_W
        pltpu.sync_copy(idx_hbm.at[pl.ds(base, _B_PER_W)], idx_v)
        pltpu.async_copy(table_hbm.at[idx_v], rows_v, sem).wait()

        @pl.loop(0, _B_PER_SUBCORE)
        def _(s):
            for t in range(_MAX_LEN):
                r = s * _MAX_LEN + t
                comp_v[s, pl.ds(t * _HIDDEN, _HALF)] = rows_v[r, pl.ds(0, _HALF)]
                comp_v[s, pl.ds(t * _HIDDEN + _HALF, _HALF)] = (
                    rows_v[r, pl.ds(_HALF, _HALF)])

        pltpu.sync_copy(comp_v, out_hbm.at[pl.ds(wid * _B_PER_SUBCORE,
                                                 _B_PER_SUBCORE)])

    return k(table_pad, idx_flat)


def _mm_body(fin_ref, w_ref, b_ref, out_hbm, flat_ref, obuf_ref, sem):
    t = pl.program_id(0)

    @pl.when(t == 0)
    def _():
        flat_ref[...] = fin_ref[...].astype(jnp.bfloat16)

    w = w_ref[0].astype(jnp.bfloat16)
    acc = lax.dot_general(
        flat_ref[...], w,
        (((1,), (1,)), ((), ())),
        preferred_element_type=jnp.float32,
    )

    slot = lax.rem(t, 2)

    @pl.when(t >= 2)
    def _():
        pltpu.make_async_copy(
            obuf_ref.at[slot], out_hbm.at[:, t - 2, :], sem.at[slot]).wait()

    obuf_ref[slot] = acc + b_ref[t, 0, :][None, :]
    pltpu.make_async_copy(
        obuf_ref.at[slot], out_hbm.at[:, t, :], sem.at[slot]).start()

    @pl.when(t == _MAX_LEN - 1)
    def _():
        pltpu.make_async_copy(
            obuf_ref.at[1 - slot], out_hbm.at[:, t - 1, :],
            sem.at[1 - slot]).wait()
        pltpu.make_async_copy(
            obuf_ref.at[slot], out_hbm.at[:, t, :], sem.at[slot]).wait()


def _projection(flat2d, W3, b3):
    return pl.pallas_call(
        _mm_body,
        grid=(_MAX_LEN,),
        in_specs=[
            pl.BlockSpec((_BATCH, _IN_F), lambda j: (0, 0)),
            pl.BlockSpec((1, _VOCAB, _IN_F), lambda j: (j, 0, 0)),
            pl.BlockSpec((_MAX_LEN, 1, _VOCAB), lambda j: (0, 0, 0)),
        ],
        out_specs=pl.BlockSpec(memory_space=pl.ANY),
        out_shape=jax.ShapeDtypeStruct((_BATCH, _MAX_LEN, _VOCAB), jnp.float32),
        scratch_shapes=[
            pltpu.VMEM((_BATCH, _IN_F), jnp.bfloat16),
            pltpu.VMEM((2, _BATCH, _VOCAB), jnp.float32),
            pltpu.SemaphoreType.DMA((2,)),
        ],
        compiler_params=pltpu.CompilerParams(
            dimension_semantics=("arbitrary",),
        ),
    )(flat2d, W3, b3)


def kernel(x, embed_table, W, b):
    idx = x.reshape(-1).astype(jnp.int32)
    table_pad = jnp.pad(embed_table, ((0, 0), (0, _PAD_W - _HIDDEN)))
    flat2d = _sc_gather(table_pad, idx)
    W3 = W.reshape(_MAX_LEN, _VOCAB, _IN_F)
    b3 = b.reshape(_MAX_LEN, 1, _VOCAB)
    return _projection(flat2d, W3, b3)


# speedup vs baseline: 1.1385x; 1.0003x over previous
import functools

import jax
import jax.numpy as jnp
from jax import lax
from jax.experimental import pallas as pl
from jax.experimental.pallas import tpu as pltpu
from jax.experimental.pallas import tpu_sc as plsc

_VOCAB = 1000
_MAX_LEN = 20
_HIDDEN = 32
_BATCH = 1024
_IN_F = _MAX_LEN * _HIDDEN
_OUT_F = _MAX_LEN * _VOCAB

_NC, _NS = 2, 16
_NW = _NC * _NS
_B_TOTAL = _BATCH * _MAX_LEN
_B_PER_W = _B_TOTAL // _NW


_PAD_W = 128


_B_PER_SUBCORE = _BATCH // _NW
_HALF = 16


def _sc_gather(table_pad, idx_flat):
    mesh = plsc.VectorSubcoreMesh(core_axis_name="c", subcore_axis_name="s")

    @functools.partial(
        pl.kernel,
        mesh=mesh,
        out_type=jax.ShapeDtypeStruct((_BATCH, _IN_F), jnp.float32),
        scratch_types=[
            pltpu.VMEM((_B_PER_W,), jnp.int32),
            pltpu.VMEM((_B_PER_W, _PAD_W), jnp.float32),
            pltpu.VMEM((_B_PER_SUBCORE, _IN_F), jnp.float32),
            pltpu.SemaphoreType.DMA,
        ],
    )
    def k(table_hbm, idx_hbm, out_hbm, idx_v, rows_v, comp_v, sem):
        wid = lax.axis_index("s") * _NC + lax.axis_index("c")
        base = wid * _B_PER_W
        pltpu.sync_copy(idx_hbm.at[pl.ds(base, _B_PER_W)], idx_v)
        pltpu.async_copy(table_hbm.at[idx_v], rows_v, sem).wait()

        @pl.loop(0, _B_PER_SUBCORE)
        def _(s):
            for t in range(_MAX_LEN):
                r = s * _MAX_LEN + t
                comp_v[s, pl.ds(t * _HIDDEN, _HALF)] = rows_v[r, pl.ds(0, _HALF)]
                comp_v[s, pl.ds(t * _HIDDEN + _HALF, _HALF)] = (
                    rows_v[r, pl.ds(_HALF, _HALF)])

        pltpu.sync_copy(comp_v, out_hbm.at[pl.ds(wid * _B_PER_SUBCORE,
                                                 _B_PER_SUBCORE)])

    return k(table_pad, idx_flat)


def _mm_body(fin_ref, w_ref, b_ref, out_hbm, flat_ref, obuf_ref, sem):
    t = pl.program_id(0)

    @pl.when(t == 0)
    def _():
        flat_ref[...] = fin_ref[...].astype(jnp.bfloat16)

    w = w_ref[...].astype(jnp.bfloat16)
    acc = lax.dot_general(
        flat_ref[...], w,
        (((1,), (1,)), ((), ())),
        preferred_element_type=jnp.float32,
    )

    slot = lax.rem(t, 2)

    @pl.when(t >= 2)
    def _():
        pltpu.make_async_copy(
            obuf_ref.at[slot], out_hbm.at[:, t - 2, :], sem.at[slot]).wait()

    obuf_ref[slot] = acc + b_ref[t, 0, :][None, :]
    pltpu.make_async_copy(
        obuf_ref.at[slot], out_hbm.at[:, t, :], sem.at[slot]).start()

    @pl.when(t == _MAX_LEN - 1)
    def _():
        pltpu.make_async_copy(
            obuf_ref.at[1 - slot], out_hbm.at[:, t - 1, :],
            sem.at[1 - slot]).wait()
        pltpu.make_async_copy(
            obuf_ref.at[slot], out_hbm.at[:, t, :], sem.at[slot]).wait()


def _projection(flat2d, W, b3):
    return pl.pallas_call(
        _mm_body,
        grid=(_MAX_LEN,),
        in_specs=[
            pl.BlockSpec((_BATCH, _IN_F), lambda j: (0, 0)),
            pl.BlockSpec((_VOCAB, _IN_F), lambda j: (j, 0)),
            pl.BlockSpec((_MAX_LEN, 1, _VOCAB), lambda j: (0, 0, 0)),
        ],
        out_specs=pl.BlockSpec(memory_space=pl.ANY),
        out_shape=jax.ShapeDtypeStruct((_BATCH, _MAX_LEN, _VOCAB), jnp.float32),
        scratch_shapes=[
            pltpu.VMEM((_BATCH, _IN_F), jnp.bfloat16),
            pltpu.VMEM((2, _BATCH, _VOCAB), jnp.float32),
            pltpu.SemaphoreType.DMA((2,)),
        ],
        compiler_params=pltpu.CompilerParams(
            dimension_semantics=("arbitrary",),
        ),
    )(flat2d, W, b3)


def kernel(x, embed_table, W, b):
    idx = x.reshape(-1).astype(jnp.int32)
    table_pad = jnp.pad(embed_table, ((0, 0), (0, _PAD_W - _HIDDEN)))
    flat2d = _sc_gather(table_pad, idx)
    b3 = b.reshape(_MAX_LEN, 1, _VOCAB)
    return _projection(flat2d, W, b3)


# speedup vs baseline: 2.2210x; 1.9508x over previous
import functools

import jax
import jax.numpy as jnp
from jax import lax
from jax.experimental import pallas as pl
from jax.experimental.pallas import tpu as pltpu
from jax.experimental.pallas import tpu_sc as plsc

_VOCAB = 1000
_MAX_LEN = 20
_HIDDEN = 32
_BATCH = 1024
_IN_F = _MAX_LEN * _HIDDEN
_OUT_F = _MAX_LEN * _VOCAB

_NC, _NS = 2, 16
_NW = _NC * _NS
_B_TOTAL = _BATCH * _MAX_LEN
_B_PER_W = _B_TOTAL // _NW


_PAD_W = 128


_B_PER_SUBCORE = _BATCH // _NW
_HALF = 16
_BT = 2


def _sc_gather(table_pad, idx_flat):
    mesh = plsc.VectorSubcoreMesh(core_axis_name="c", subcore_axis_name="s")

    @functools.partial(
        pl.kernel,
        mesh=mesh,
        out_type=jax.ShapeDtypeStruct((_BATCH, _IN_F), jnp.float32),
        scratch_types=[
            pltpu.VMEM((_B_PER_W,), jnp.int32),
            pltpu.VMEM((_B_PER_W, _PAD_W), jnp.float32),
            pltpu.VMEM((_B_PER_SUBCORE, _IN_F), jnp.float32),
            pltpu.SemaphoreType.DMA,
        ],
    )
    def k(table_hbm, idx_hbm, out_hbm, idx_v, rows_v, comp_v, sem):
        wid = lax.axis_index("s") * _NC + lax.axis_index("c")
        base = wid * _B_PER_W
        pltpu.sync_copy(idx_hbm.at[pl.ds(base, _B_PER_W)], idx_v)
        pltpu.async_copy(table_hbm.at[idx_v], rows_v, sem).wait()

        @pl.loop(0, _B_PER_SUBCORE)
        def _(s):
            for t in range(_MAX_LEN):
                r = s * _MAX_LEN + t
                comp_v[s, pl.ds(t * _HIDDEN, _HALF)] = rows_v[r, pl.ds(0, _HALF)]
                comp_v[s, pl.ds(t * _HIDDEN + _HALF, _HALF)] = (
                    rows_v[r, pl.ds(_HALF, _HALF)])

        pltpu.sync_copy(comp_v, out_hbm.at[pl.ds(wid * _B_PER_SUBCORE,
                                                 _B_PER_SUBCORE)])

    return k(table_pad, idx_flat)


def _mm_body(fin_ref, w_ref, b_ref, out_ref, flat_ref):
    t = pl.program_id(0)

    @pl.when(t == 0)
    def _():
        flat_ref[...] = fin_ref[...].astype(jnp.bfloat16)

    w = w_ref[...].astype(jnp.bfloat16)
    acc = lax.dot_general(
        w, flat_ref[...],
        (((1,), (1,)), ((), ())),
        preferred_element_type=jnp.float32,
    )
    for k in range(_BT):
        out_ref[k] = (acc[k * _VOCAB:(k + 1) * _VOCAB, :]
                      + b_ref[k])


def _projection(flat2d, W, b3):
    return pl.pallas_call(
        _mm_body,
        grid=(_MAX_LEN // _BT,),
        in_specs=[
            pl.BlockSpec((_BATCH, _IN_F), lambda j: (0, 0)),
            pl.BlockSpec((_BT * _VOCAB, _IN_F), lambda j: (j, 0)),
            pl.BlockSpec((_BT, _VOCAB, 1), lambda j: (j, 0, 0)),
        ],
        out_specs=pl.BlockSpec((_BT, _VOCAB, _BATCH), lambda j: (j, 0, 0)),
        out_shape=jax.ShapeDtypeStruct((_MAX_LEN, _VOCAB, _BATCH), jnp.float32),
        scratch_shapes=[pltpu.VMEM((_BATCH, _IN_F), jnp.bfloat16)],
        compiler_params=pltpu.CompilerParams(
            dimension_semantics=("arbitrary",),
        ),
    )(flat2d, W, b3)


def kernel(x, embed_table, W, b):
    idx = x.reshape(-1).astype(jnp.int32)
    table_pad = jnp.pad(embed_table, ((0, 0), (0, _PAD_W - _HIDDEN)))
    flat2d = _sc_gather(table_pad, idx)
    b3 = b.reshape(_MAX_LEN, _VOCAB, 1)
    out_t = _projection(flat2d, W, b3)
    return out_t.transpose(2, 0, 1)


# speedup vs baseline: 2.3652x; 1.0649x over previous
import functools

import jax
import jax.numpy as jnp
from jax import lax
from jax.experimental import pallas as pl
from jax.experimental.pallas import tpu as pltpu
from jax.experimental.pallas import tpu_sc as plsc

_VOCAB = 1000
_MAX_LEN = 20
_HIDDEN = 32
_BATCH = 1024
_IN_F = _MAX_LEN * _HIDDEN
_OUT_F = _MAX_LEN * _VOCAB

_NC, _NS = 2, 16
_NW = _NC * _NS
_B_TOTAL = _BATCH * _MAX_LEN
_B_PER_W = _B_TOTAL // _NW


_PAD_W = 128


_B_PER_SUBCORE = _BATCH // _NW
_HALF = 16
_BT = 2


def _sc_gather(table_pad, idx_flat):
    mesh = plsc.VectorSubcoreMesh(core_axis_name="c", subcore_axis_name="s")

    @functools.partial(
        pl.kernel,
        mesh=mesh,
        out_type=jax.ShapeDtypeStruct((_BATCH, _IN_F), jnp.float32),
        scratch_types=[
            pltpu.VMEM((_B_PER_W,), jnp.int32),
            pltpu.VMEM((_B_PER_W, _PAD_W), jnp.float32),
            pltpu.VMEM((_B_PER_SUBCORE, _IN_F), jnp.float32),
            pltpu.SemaphoreType.DMA,
        ],
    )
    def k(table_hbm, idx_hbm, out_hbm, idx_v, rows_v, comp_v, sem):
        wid = lax.axis_index("s") * _NC + lax.axis_index("c")
        b0 = wid * _B_PER_SUBCORE
        base = wid * _B_PER_W
        pltpu.sync_copy(idx_hbm.at[pl.ds(base, _B_PER_W)], idx_v)
        pltpu.async_copy(table_hbm.at[idx_v], rows_v, sem).wait()

        @pl.loop(0, _B_PER_SUBCORE)
        def _(s):
            for t in range(_MAX_LEN):
                r = s * _MAX_LEN + t
                comp_v[s, pl.ds(t * _HIDDEN, _HALF)] = rows_v[r, pl.ds(0, _HALF)]
                comp_v[s, pl.ds(t * _HIDDEN + _HALF, _HALF)] = (
                    rows_v[r, pl.ds(_HALF, _HALF)])

        pltpu.sync_copy(comp_v, out_hbm.at[pl.ds(b0, _B_PER_SUBCORE)])

    return k(table_pad, idx_flat)


def _mm_body(fin_ref, w_ref, b_ref, out_ref, flat_ref):
    t = pl.program_id(0)

    @pl.when(t == 0)
    def _():
        flat_ref[...] = fin_ref[...].astype(jnp.bfloat16)

    w = w_ref[...].astype(jnp.bfloat16)
    acc = lax.dot_general(
        w, flat_ref[...],
        (((1,), (1,)), ((), ())),
        preferred_element_type=jnp.float32,
    )
    for k in range(_BT):
        bcol = b_ref[k].reshape(_VOCAB, 1)
        out_ref[k] = acc[k * _VOCAB:(k + 1) * _VOCAB, :] + bcol


def _projection(flat2d, W, b3):
    return pl.pallas_call(
        _mm_body,
        grid=(_MAX_LEN // _BT,),
        in_specs=[
            pl.BlockSpec((_BATCH, _IN_F), lambda j: (0, 0)),
            pl.BlockSpec((_BT * _VOCAB, _IN_F), lambda j: (j, 0)),
            pl.BlockSpec((_BT, 1, _VOCAB), lambda j: (j, 0, 0)),
        ],
        out_specs=pl.BlockSpec((_BT, _VOCAB, _BATCH), lambda j: (j, 0, 0)),
        out_shape=jax.ShapeDtypeStruct((_MAX_LEN, _VOCAB, _BATCH), jnp.float32),
        scratch_shapes=[pltpu.VMEM((_BATCH, _IN_F), jnp.bfloat16)],
        compiler_params=pltpu.CompilerParams(
            dimension_semantics=("arbitrary",),
        ),
    )(flat2d, W, b3)


def kernel(x, embed_table, W, b):
    idx = x.reshape(-1).astype(jnp.int32)
    table_pad = jnp.pad(embed_table, ((0, 0), (0, _PAD_W - _HIDDEN)))
    flat2d = _sc_gather(table_pad, idx)
    b3 = b.reshape(_MAX_LEN, 1, _VOCAB)
    out_t = _projection(flat2d, W, b3)
    return out_t.transpose(2, 0, 1)


# speedup vs baseline: 2.4362x; 1.0300x over previous
import functools

import jax
import jax.numpy as jnp
from jax import lax
from jax.experimental import pallas as pl
from jax.experimental.pallas import tpu as pltpu
from jax.experimental.pallas import tpu_sc as plsc

_VOCAB = 1000
_MAX_LEN = 20
_HIDDEN = 32
_BATCH = 1024
_IN_F = _MAX_LEN * _HIDDEN
_OUT_F = _MAX_LEN * _VOCAB

_NC, _NS = 2, 16
_NW = _NC * _NS
_B_TOTAL = _BATCH * _MAX_LEN
_B_PER_W = _B_TOTAL // _NW


_PAD_W = 128


_B_PER_SUBCORE = _BATCH // _NW
_HALF = 16
_BT = 2


def _sc_gather(table_pad, idx_flat):
    mesh = plsc.VectorSubcoreMesh(core_axis_name="c", subcore_axis_name="s")

    @functools.partial(
        pl.kernel,
        mesh=mesh,
        out_type=jax.ShapeDtypeStruct((_BATCH, _IN_F), jnp.float32),
        scratch_types=[
            pltpu.VMEM((_B_PER_W,), jnp.int32),
            pltpu.VMEM((_B_PER_W, _PAD_W), jnp.float32),
            pltpu.VMEM((_B_PER_SUBCORE, _IN_F), jnp.float32),
            pltpu.VMEM_SHARED((_VOCAB, _PAD_W), jnp.float32),
            pltpu.SemaphoreType.DMA,
        ],
    )
    def k(table_hbm, idx_hbm, out_hbm, idx_v, rows_v, comp_v, stab, sem):
        sid = lax.axis_index("s")
        wid = sid * _NC + lax.axis_index("c")
        b0 = wid * _B_PER_SUBCORE
        base = wid * _B_PER_W

        @pl.when(sid == 0)
        def _():
            pltpu.sync_copy(table_hbm, stab)

        pltpu.sync_copy(idx_hbm.at[pl.ds(base, _B_PER_W)], idx_v)
        plsc.subcore_barrier()
        pltpu.async_copy(stab.at[idx_v], rows_v, sem).wait()

        @pl.loop(0, _B_PER_SUBCORE)
        def _(s):
            for t in range(_MAX_LEN):
                r = s * _MAX_LEN + t
                comp_v[s, pl.ds(t * _HIDDEN, _HALF)] = rows_v[r, pl.ds(0, _HALF)]
                comp_v[s, pl.ds(t * _HIDDEN + _HALF, _HALF)] = (
                    rows_v[r, pl.ds(_HALF, _HALF)])

        pltpu.sync_copy(comp_v, out_hbm.at[pl.ds(b0, _B_PER_SUBCORE)])

    return k(table_pad, idx_flat)


def _mm_body(fin_ref, w_ref, b_ref, out_ref, flat_ref):
    t = pl.program_id(0)

    @pl.when(t == 0)
    def _():
        flat_ref[...] = fin_ref[...].astype(jnp.bfloat16)

    w = w_ref[...].astype(jnp.bfloat16)
    acc = lax.dot_general(
        w, flat_ref[...],
        (((1,), (1,)), ((), ())),
        preferred_element_type=jnp.float32,
    )
    for k in range(_BT):
        bcol = b_ref[k].reshape(_VOCAB, 1)
        out_ref[k] = acc[k * _VOCAB:(k + 1) * _VOCAB, :] + bcol


def _projection(flat2d, W, b3):
    return pl.pallas_call(
        _mm_body,
        grid=(_MAX_LEN // _BT,),
        in_specs=[
            pl.BlockSpec((_BATCH, _IN_F), lambda j: (0, 0)),
            pl.BlockSpec((_BT * _VOCAB, _IN_F), lambda j: (j, 0)),
            pl.BlockSpec((_BT, 1, _VOCAB), lambda j: (j, 0, 0)),
        ],
        out_specs=pl.BlockSpec((_BT, _VOCAB, _BATCH), lambda j: (j, 0, 0)),
        out_shape=jax.ShapeDtypeStruct((_MAX_LEN, _VOCAB, _BATCH), jnp.float32),
        scratch_shapes=[pltpu.VMEM((_BATCH, _IN_F), jnp.bfloat16)],
        compiler_params=pltpu.CompilerParams(
            dimension_semantics=("arbitrary",),
        ),
    )(flat2d, W, b3)


def kernel(x, embed_table, W, b):
    idx = x.reshape(-1).astype(jnp.int32)
    table_pad = jnp.pad(embed_table, ((0, 0), (0, _PAD_W - _HIDDEN)))
    flat2d = _sc_gather(table_pad, idx)
    b3 = b.reshape(_MAX_LEN, 1, _VOCAB)
    out_t = _projection(flat2d, W, b3)
    return out_t.transpose(2, 0, 1)


# speedup vs baseline: 2.4633x; 1.0111x over previous
import functools

import jax
import jax.numpy as jnp
from jax import lax
from jax.experimental import pallas as pl
from jax.experimental.pallas import tpu as pltpu
from jax.experimental.pallas import tpu_sc as plsc

_VOCAB = 1000
_MAX_LEN = 20
_HIDDEN = 32
_BATCH = 1024
_IN_F = _MAX_LEN * _HIDDEN
_OUT_F = _MAX_LEN * _VOCAB

_NC, _NS = 2, 16
_NW = _NC * _NS
_B_TOTAL = _BATCH * _MAX_LEN
_B_PER_W = _B_TOTAL // _NW


_PAD_W = 128


_B_PER_SUBCORE = _BATCH // _NW
_HALF = 16
_BT = 2


def _sc_gather(table_pad, idx_flat):
    mesh = plsc.VectorSubcoreMesh(core_axis_name="c", subcore_axis_name="s")

    @functools.partial(
        pl.kernel,
        mesh=mesh,
        out_type=jax.ShapeDtypeStruct((_BATCH, _IN_F), jnp.float32),
        scratch_types=[
            pltpu.VMEM((_B_PER_W,), jnp.int32),
            pltpu.VMEM((_B_PER_W, _PAD_W), jnp.float32),
            pltpu.VMEM((_B_PER_SUBCORE, _IN_F), jnp.float32),
            pltpu.VMEM_SHARED((_VOCAB, _PAD_W), jnp.float32),
            pltpu.SemaphoreType.DMA,
        ],
    )
    def k(table_hbm, xt_hbm, out_hbm, idx_v, rows_v, comp_v, stab, sem):
        sid = lax.axis_index("s")
        wid = sid * _NC + lax.axis_index("c")
        b0 = wid * _B_PER_SUBCORE

        @pl.when(sid == 0)
        def _():
            pltpu.sync_copy(table_hbm, stab)

        idx_copies = [
            pltpu.async_copy(
                xt_hbm.at[t, pl.ds(b0, _B_PER_SUBCORE)],
                idx_v.at[pl.ds(t * _B_PER_SUBCORE, _B_PER_SUBCORE)], sem)
            for t in range(_MAX_LEN)
        ]
        for c in idx_copies:
            c.wait()
        plsc.subcore_barrier()
        pltpu.async_copy(stab.at[idx_v], rows_v, sem).wait()

        @pl.loop(0, _B_PER_SUBCORE)
        def _(s):
            for t in range(_MAX_LEN):
                r = t * _B_PER_SUBCORE + s
                comp_v[s, pl.ds(t * _HIDDEN, _HALF)] = rows_v[r, pl.ds(0, _HALF)]
                comp_v[s, pl.ds(t * _HIDDEN + _HALF, _HALF)] = (
                    rows_v[r, pl.ds(_HALF, _HALF)])

        pltpu.sync_copy(comp_v, out_hbm.at[pl.ds(b0, _B_PER_SUBCORE)])

    return k(table_pad, idx_flat)


def _mm_body(fin_ref, w_ref, b_ref, out_ref, flat_ref):
    t = pl.program_id(0)

    @pl.when(t == 0)
    def _():
        flat_ref[...] = fin_ref[...].astype(jnp.bfloat16)

    w = w_ref[...].astype(jnp.bfloat16)
    acc = lax.dot_general(
        w, flat_ref[...],
        (((1,), (1,)), ((), ())),
        preferred_element_type=jnp.float32,
    )
    for k in range(_BT):
        bcol = b_ref[k].reshape(_VOCAB, 1)
        out_ref[k] = acc[k * _VOCAB:(k + 1) * _VOCAB, :] + bcol


def _projection(flat2d, W, b3):
    return pl.pallas_call(
        _mm_body,
        grid=(_MAX_LEN // _BT,),
        in_specs=[
            pl.BlockSpec((_BATCH, _IN_F), lambda j: (0, 0)),
            pl.BlockSpec((_BT * _VOCAB, _IN_F), lambda j: (j, 0)),
            pl.BlockSpec((_BT, 1, _VOCAB), lambda j: (j, 0, 0)),
        ],
        out_specs=pl.BlockSpec((_BT, _VOCAB, _BATCH), lambda j: (j, 0, 0)),
        out_shape=jax.ShapeDtypeStruct((_MAX_LEN, _VOCAB, _BATCH), jnp.float32),
        scratch_shapes=[pltpu.VMEM((_BATCH, _IN_F), jnp.bfloat16)],
        compiler_params=pltpu.CompilerParams(
            dimension_semantics=("arbitrary",),
        ),
    )(flat2d, W, b3)


def kernel(x, embed_table, W, b):
    idx = x.T.astype(jnp.int32)
    table_pad = jnp.pad(embed_table, ((0, 0), (0, _PAD_W - _HIDDEN)))
    flat2d = _sc_gather(table_pad, idx)
    b3 = b.reshape(_MAX_LEN, 1, _VOCAB)
    out_t = _projection(flat2d, W, b3)
    return out_t.transpose(2, 0, 1)
